# baseline (speedup 1.0000x reference)
import functools

import jax
import jax.numpy as jnp
from jax import lax
from jax.experimental import pallas as pl
from jax.experimental.pallas import tpu as pltpu
from jax.experimental.pallas import tpu_sc as plsc

N = 10000
D = 128
H = 64
E = 320000

NC, NS, LANES = 2, 16, 16
NW = NC * NS

NPAD = 10240
NJUNK = NPAD - N
EPAD = 327680
BATCH = 128
NB = EPAD // (NW * BATCH)
CK = 4
RPT = NPAD // NS
ZR = 64

_MESH = plsc.VectorSubcoreMesh(
    core_axis_name="c", subcore_axis_name="s", num_cores=NC, num_subcores=NS)
_SC_PARAMS = pltpu.CompilerParams(use_tc_tiling_on_sc=False)


def _count_body(eip_hbm, cnt_hbm, cnt_sh, idx_v, ones_v, zvec_v, sem):
    c = lax.axis_index("c")
    s = lax.axis_index("s")
    wid = c * NS + s
    for i in range(BATCH // LANES):
        ones_v[pl.ds(i * LANES, LANES)] = jnp.ones((LANES,), jnp.float32)
    for i in range(RPT // LANES):
        zvec_v[pl.ds(i * LANES, LANES)] = jnp.zeros((LANES,), jnp.float32)
    pltpu.sync_copy(zvec_v, cnt_sh.at[pl.ds(s * RPT, RPT)])
    plsc.subcore_barrier()
    pltpu.sync_copy(eip_hbm.at[1, pl.ds(wid * NB, NB)], idx_v)
    for g in range(NB // 16):
        descs = [
            pltpu.async_copy(ones_v, cnt_sh.at[idx_v.at[16 * g + j]], sem,
                             add=True)
            for j in range(16)
        ]
        for dd in descs:
            dd.wait()
    plsc.subcore_barrier()
    pltpu.sync_copy(cnt_sh.at[pl.ds(s * RPT, RPT)],
                    cnt_hbm.at[c, pl.ds(s * RPT, RPT)])


_count_edges = functools.partial(
    pl.kernel,
    _count_body,
    out_type=jax.ShapeDtypeStruct((NC, NPAD), jnp.float32),
    mesh=_MESH,
    scratch_types=[
        pltpu.VMEM_SHARED((NPAD,), jnp.float32),
        pltpu.VMEM((NB, BATCH), jnp.int32),
        pltpu.VMEM((BATCH,), jnp.float32),
        pltpu.VMEM((RPT,), jnp.float32),
        pltpu.SemaphoreType.DMA,
    ],
    compiler_params=_SC_PARAMS,
)()


def _scatter_body(eip_hbm, y_hbm, agg_hbm,
                  agg_sh, isrc_v, idst_v, rows_v, zbuf_v, gsem, ssem):
    c = lax.axis_index("c")
    s = lax.axis_index("s")
    wid = c * NS + s
    for r in range(ZR):
        for k in range(H // LANES):
            zbuf_v[r, pl.ds(k * LANES, LANES)] = jnp.zeros((LANES,), jnp.float32)
    for t in range(RPT // ZR):
        pltpu.sync_copy(zbuf_v, agg_sh.at[pl.ds(s * RPT + t * ZR, ZR)])
    plsc.subcore_barrier()
    pltpu.sync_copy(eip_hbm.at[0, pl.ds(wid * NB, NB)], isrc_v)
    pltpu.sync_copy(eip_hbm.at[1, pl.ds(wid * NB, NB)], idst_v)
    ncH = NB // CK

    def _fire_gather(t, buf):
        return [
            pltpu.async_copy(y_hbm.at[isrc_v.at[CK * t + j]],
                             rows_v.at[buf, j], gsem)
            for j in range(CK)
        ]

    def _fire_scatter(t, buf):
        return [
            pltpu.async_copy(rows_v.at[buf, j],
                             agg_sh.at[idst_v.at[CK * t + j]], ssem, add=True)
            for j in range(CK)
        ]

    gd = _fire_gather(0, 0)
    sd = []
    for t in range(ncH):
        p = t % 2
        for dd in gd:
            dd.wait()
        for dd in sd:
            dd.wait()
        gd = _fire_gather(t + 1, 1 - p) if t + 1 < ncH else []
        sd = _fire_scatter(t, p)
    for dd in sd:
        dd.wait()
    plsc.subcore_barrier()
    pltpu.sync_copy(agg_sh.at[pl.ds(s * RPT, RPT)],
                    agg_hbm.at[c, pl.ds(s * RPT, RPT)])


_scatter_edges = functools.partial(
    pl.kernel,
    _scatter_body,
    out_type=jax.ShapeDtypeStruct((NC, NPAD, H), jnp.float32),
    mesh=_MESH,
    scratch_types=[
        pltpu.VMEM_SHARED((NPAD, H), jnp.float32),
        pltpu.VMEM((NB, BATCH), jnp.int32),
        pltpu.VMEM((NB, BATCH), jnp.int32),
        pltpu.VMEM((2, CK, BATCH, H), jnp.float32),
        pltpu.VMEM((ZR, H), jnp.float32),
        pltpu.SemaphoreType.DMA,
        pltpu.SemaphoreType.DMA,
    ],
    compiler_params=_SC_PARAMS,
)()


BM = 1024


def _xw_body(x_ref, w_ref, y_ref):
    y_ref[...] = jnp.dot(x_ref[...], w_ref[...],
                         preferred_element_type=jnp.float32)


def _y1_body(xw_ref, cnt_ref, y_ref):
    dinv = lax.rsqrt(1.0 + cnt_ref[0, :] + cnt_ref[1, :])
    y_ref[...] = xw_ref[...] * dinv[:, None]


def _mid_body(agg_ref, y_ref, cnt_ref, b_ref, w_ref, out_ref):
    dinv = lax.rsqrt(1.0 + cnt_ref[0, :] + cnt_ref[1, :])
    pre = (agg_ref[0] + agg_ref[1] + y_ref[...]) * dinv[:, None] + b_ref[...]
    h = jnp.maximum(pre, 0.0)
    hw = jnp.dot(h, w_ref[...], preferred_element_type=jnp.float32)
    out_ref[...] = hw * dinv[:, None]


def _fin_body(agg_ref, y_ref, cnt_ref, b_ref, wfc_ref, bfc_ref, out_ref):
    dinv = lax.rsqrt(1.0 + cnt_ref[0, :] + cnt_ref[1, :])
    pre = (agg_ref[0] + agg_ref[1] + y_ref[...]) * dinv[:, None] + b_ref[...]
    h = jnp.maximum(pre, 0.0)
    out_ref[...] = (jnp.dot(h, wfc_ref[...], preferred_element_type=jnp.float32)
                    + bfc_ref[...])


def _tc_xw(x, W1):
    return pl.pallas_call(
        _xw_body,
        grid=(NPAD // BM,),
        in_specs=[
            pl.BlockSpec((BM, D), lambda i: (i, 0)),
            pl.BlockSpec((D, H), lambda i: (0, 0)),
        ],
        out_specs=pl.BlockSpec((BM, H), lambda i: (i, 0)),
        out_shape=jax.ShapeDtypeStruct((N, H), jnp.float32),
    )(x, W1)


def _tc_y1(xw, cnt):
    return pl.pallas_call(
        _y1_body,
        grid=(NPAD // BM,),
        in_specs=[
            pl.BlockSpec((BM, H), lambda i: (i, 0)),
            pl.BlockSpec((NC, BM), lambda i: (0, i)),
        ],
        out_specs=pl.BlockSpec((BM, H), lambda i: (i, 0)),
        out_shape=jax.ShapeDtypeStruct((N, H), jnp.float32),
    )(xw, cnt)


def _tc_mid(agg, y1, cnt, b1, W2):
    return pl.pallas_call(
        _mid_body,
        grid=(NPAD // BM,),
        in_specs=[
            pl.BlockSpec((NC, BM, H), lambda i: (0, i, 0)),
            pl.BlockSpec((BM, H), lambda i: (i, 0)),
            pl.BlockSpec((NC, BM), lambda i: (0, i)),
            pl.BlockSpec((1, H), lambda i: (0, 0)),
            pl.BlockSpec((H, H), lambda i: (0, 0)),
        ],
        out_specs=pl.BlockSpec((BM, H), lambda i: (i, 0)),
        out_shape=jax.ShapeDtypeStruct((N, H), jnp.float32),
    )(agg, y1, cnt, b1.reshape(1, H), W2)


def _tc_fin(agg, y2, cnt, b2, Wfc, bfc):
    return pl.pallas_call(
        _fin_body,
        grid=(NPAD // BM,),
        in_specs=[
            pl.BlockSpec((NC, BM, H), lambda i: (0, i, 0)),
            pl.BlockSpec((BM, H), lambda i: (i, 0)),
            pl.BlockSpec((NC, BM), lambda i: (0, i)),
            pl.BlockSpec((1, H), lambda i: (0, 0)),
            pl.BlockSpec((H, H), lambda i: (0, 0)),
            pl.BlockSpec((1, H), lambda i: (0, 0)),
        ],
        out_specs=pl.BlockSpec((BM, H), lambda i: (i, 0)),
        out_shape=jax.ShapeDtypeStruct((N, H), jnp.float32),
    )(agg, y2, cnt, b2.reshape(1, H), Wfc, bfc.reshape(1, H))


def kernel(x, edge_index, W1, b1, W2, b2, Wfc, bfc):
    ei = edge_index.astype(jnp.int32)
    spread = jnp.arange(EPAD - E, dtype=jnp.int32) % NJUNK
    pad2 = jnp.stack([spread, N + spread])
    eip = jnp.concatenate([ei, pad2], axis=1).reshape(2, EPAD // BATCH, BATCH)

    cnt = _count_edges(eip)
    xw1 = _tc_xw(x, W1)
    y1 = _tc_y1(xw1, cnt)
    agg1 = _scatter_edges(eip, y1)
    y2 = _tc_mid(agg1, y1, cnt, b1, W2)
    agg2 = _scatter_edges(eip, y2)
    return _tc_fin(agg2, y2, cnt, b2, Wfc, bfc)


# speedup vs baseline: 1.0584x; 1.0584x over previous
import functools

import jax
import jax.numpy as jnp
from jax import lax
from jax.experimental import pallas as pl
from jax.experimental.pallas import tpu as pltpu
from jax.experimental.pallas import tpu_sc as plsc

N = 10000
D = 128
H = 64
E = 320000

NC, NS, LANES = 2, 16, 16
NW = NC * NS

NPAD = 10240
NJUNK = NPAD - N
EPAD = 327680
BATCH = 128
NB = EPAD // (NW * BATCH)
CK = 4
RPT = NPAD // NS
ZR = 64

_MESH = plsc.VectorSubcoreMesh(
    core_axis_name="c", subcore_axis_name="s", num_cores=NC, num_subcores=NS)
_SC_PARAMS = pltpu.CompilerParams(use_tc_tiling_on_sc=False)


def _count_body(eip_hbm, cnt_hbm, cnt_sh, idx_v, ones_v, zvec_v, sem):
    c = lax.axis_index("c")
    s = lax.axis_index("s")
    wid = c * NS + s
    for i in range(BATCH // LANES):
        ones_v[pl.ds(i * LANES, LANES)] = jnp.ones((LANES,), jnp.float32)
    for i in range(RPT // LANES):
        zvec_v[pl.ds(i * LANES, LANES)] = jnp.zeros((LANES,), jnp.float32)
    pltpu.sync_copy(zvec_v, cnt_sh.at[pl.ds(s * RPT, RPT)])
    plsc.subcore_barrier()
    pltpu.sync_copy(eip_hbm.at[1, pl.ds(wid * NB, NB)], idx_v)
    for g in range(NB // 16):
        descs = [
            pltpu.async_copy(ones_v, cnt_sh.at[idx_v.at[16 * g + j]], sem,
                             add=True)
            for j in range(16)
        ]
        for dd in descs:
            dd.wait()
    plsc.subcore_barrier()
    pltpu.sync_copy(cnt_sh.at[pl.ds(s * RPT, RPT)],
                    cnt_hbm.at[c, pl.ds(s * RPT, RPT)])


_count_edges = functools.partial(
    pl.kernel,
    _count_body,
    out_type=jax.ShapeDtypeStruct((NC, NPAD), jnp.float32),
    mesh=_MESH,
    scratch_types=[
        pltpu.VMEM_SHARED((NPAD,), jnp.float32),
        pltpu.VMEM((NB, BATCH), jnp.int32),
        pltpu.VMEM((BATCH,), jnp.float32),
        pltpu.VMEM((RPT,), jnp.float32),
        pltpu.SemaphoreType.DMA,
    ],
    compiler_params=_SC_PARAMS,
)()


def _scatter_body(eip_hbm, y_hbm, agg_hbm,
                  agg_sh, isrc_v, idst_v, rows_v, zbuf_v, gsem, ssem):
    c = lax.axis_index("c")
    s = lax.axis_index("s")
    wid = c * NS + s
    for r in range(ZR):
        for k in range(H // LANES):
            zbuf_v[r, pl.ds(k * LANES, LANES)] = jnp.zeros((LANES,), jnp.float32)
    for t in range(RPT // ZR):
        pltpu.sync_copy(zbuf_v, agg_sh.at[pl.ds(s * RPT + t * ZR, ZR)])
    plsc.subcore_barrier()
    pltpu.sync_copy(eip_hbm.at[0, pl.ds(wid * NB, NB)], isrc_v)
    pltpu.sync_copy(eip_hbm.at[1, pl.ds(wid * NB, NB)], idst_v)
    ncH = NB // CK

    def _fire_gather(t, buf):
        return [
            pltpu.async_copy(y_hbm.at[isrc_v.at[CK * t + j]],
                             rows_v.at[buf, j], gsem)
            for j in range(CK)
        ]

    def _fire_scatter(t, buf):
        return [
            pltpu.async_copy(rows_v.at[buf, j],
                             agg_sh.at[idst_v.at[CK * t + j]], ssem, add=True)
            for j in range(CK)
        ]

    gd = _fire_gather(0, 0)
    sd = []
    for t in range(ncH):
        p = t % 2
        for dd in gd:
            dd.wait()
        for dd in sd:
            dd.wait()
        gd = _fire_gather(t + 1, 1 - p) if t + 1 < ncH else []
        sd = _fire_scatter(t, p)
    for dd in sd:
        dd.wait()
    plsc.subcore_barrier()
    pltpu.sync_copy(agg_sh.at[pl.ds(s * RPT, RPT)],
                    agg_hbm.at[c, pl.ds(s * RPT, RPT)])


_scatter_edges = functools.partial(
    pl.kernel,
    _scatter_body,
    out_type=jax.ShapeDtypeStruct((NC, NPAD, H), jnp.float32),
    mesh=_MESH,
    scratch_types=[
        pltpu.VMEM_SHARED((NPAD, H), jnp.float32),
        pltpu.VMEM((NB, BATCH), jnp.int32),
        pltpu.VMEM((NB, BATCH), jnp.int32),
        pltpu.VMEM((2, CK, BATCH, H), jnp.float32),
        pltpu.VMEM((ZR, H), jnp.float32),
        pltpu.SemaphoreType.DMA,
        pltpu.SemaphoreType.DMA,
    ],
    compiler_params=_SC_PARAMS,
)()


BM = 2560


def _y1_body(x_ref, w_ref, cnt_ref, y_ref):
    dinv = lax.rsqrt(1.0 + cnt_ref[0, :] + cnt_ref[1, :])
    xw = jnp.dot(x_ref[...], w_ref[...], preferred_element_type=jnp.float32)
    y_ref[...] = xw * dinv[:, None]


def _mid_body(agg_ref, y_ref, cnt_ref, b_ref, w_ref, out_ref):
    dinv = lax.rsqrt(1.0 + cnt_ref[0, :] + cnt_ref[1, :])
    pre = (agg_ref[0] + agg_ref[1] + y_ref[...]) * dinv[:, None] + b_ref[...]
    h = jnp.maximum(pre, 0.0)
    hw = jnp.dot(h, w_ref[...], preferred_element_type=jnp.float32)
    out_ref[...] = hw * dinv[:, None]


def _fin_body(agg_ref, y_ref, cnt_ref, b_ref, wfc_ref, bfc_ref, out_ref):
    dinv = lax.rsqrt(1.0 + cnt_ref[0, :] + cnt_ref[1, :])
    pre = (agg_ref[0] + agg_ref[1] + y_ref[...]) * dinv[:, None] + b_ref[...]
    h = jnp.maximum(pre, 0.0)
    out_ref[...] = (jnp.dot(h, wfc_ref[...], preferred_element_type=jnp.float32)
                    + bfc_ref[...])


def _tc_y1(x, W1, cnt):
    return pl.pallas_call(
        _y1_body,
        grid=(NPAD // BM,),
        in_specs=[
            pl.BlockSpec((BM, D), lambda i: (i, 0)),
            pl.BlockSpec((D, H), lambda i: (0, 0)),
            pl.BlockSpec((NC, BM), lambda i: (0, i)),
        ],
        out_specs=pl.BlockSpec((BM, H), lambda i: (i, 0)),
        out_shape=jax.ShapeDtypeStruct((N, H), jnp.float32),
    )(x, W1, cnt)


def _tc_mid(agg, y1, cnt, b1, W2):
    return pl.pallas_call(
        _mid_body,
        grid=(NPAD // BM,),
        in_specs=[
            pl.BlockSpec((NC, BM, H), lambda i: (0, i, 0)),
            pl.BlockSpec((BM, H), lambda i: (i, 0)),
            pl.BlockSpec((NC, BM), lambda i: (0, i)),
            pl.BlockSpec((1, H), lambda i: (0, 0)),
            pl.BlockSpec((H, H), lambda i: (0, 0)),
        ],
        out_specs=pl.BlockSpec((BM, H), lambda i: (i, 0)),
        out_shape=jax.ShapeDtypeStruct((N, H), jnp.float32),
    )(agg, y1, cnt, b1.reshape(1, H), W2)


def _tc_fin(agg, y2, cnt, b2, Wfc, bfc):
    return pl.pallas_call(
        _fin_body,
        grid=(NPAD // BM,),
        in_specs=[
            pl.BlockSpec((NC, BM, H), lambda i: (0, i, 0)),
            pl.BlockSpec((BM, H), lambda i: (i, 0)),
            pl.BlockSpec((NC, BM), lambda i: (0, i)),
            pl.BlockSpec((1, H), lambda i: (0, 0)),
            pl.BlockSpec((H, H), lambda i: (0, 0)),
            pl.BlockSpec((1, H), lambda i: (0, 0)),
        ],
        out_specs=pl.BlockSpec((BM, H), lambda i: (i, 0)),
        out_shape=jax.ShapeDtypeStruct((N, H), jnp.float32),
    )(agg, y2, cnt, b2.reshape(1, H), Wfc, bfc.reshape(1, H))


def kernel(x, edge_index, W1, b1, W2, b2, Wfc, bfc):
    ei = edge_index.astype(jnp.int32)
    spread = jnp.arange(EPAD - E, dtype=jnp.int32) % NJUNK
    pad2 = jnp.stack([spread, N + spread])
    eip = jnp.concatenate([ei, pad2], axis=1).reshape(2, EPAD // BATCH, BATCH)

    cnt = _count_edges(eip)
    y1 = _tc_y1(x, W1, cnt)
    agg1 = _scatter_edges(eip, y1)
    y2 = _tc_mid(agg1, y1, cnt, b1, W2)
    agg2 = _scatter_edges(eip, y2)
    return _tc_fin(agg2, y2, cnt, b2, Wfc, bfc)


# speedup vs baseline: 1.0872x; 1.0272x over previous
import functools

import jax
import jax.numpy as jnp
from jax import lax
from jax.experimental import pallas as pl
from jax.experimental.pallas import tpu as pltpu
from jax.experimental.pallas import tpu_sc as plsc

N = 10000
D = 128
H = 64
E = 320000

NC, NS, LANES = 2, 16, 16
NW = NC * NS

NPAD = 10240
NJUNK = NPAD - N
EPAD = 327680
BATCH = 128
NB = EPAD // (NW * BATCH)
CK = 4
RPT = NPAD // NS
ZR = 64

_MESH = plsc.VectorSubcoreMesh(
    core_axis_name="c", subcore_axis_name="s", num_cores=NC, num_subcores=NS)
_SC_PARAMS = pltpu.CompilerParams(use_tc_tiling_on_sc=False)


def _count_body(eip_hbm, cnt_hbm, cnt_sh, idx_v, ones_v, zvec_v, sem):
    c = lax.axis_index("c")
    s = lax.axis_index("s")
    wid = c * NS + s
    ld = pltpu.async_copy(eip_hbm.at[1, pl.ds(wid * NB, NB)], idx_v, sem)
    for i in range(BATCH // LANES):
        ones_v[pl.ds(i * LANES, LANES)] = jnp.ones((LANES,), jnp.float32)
    for i in range(RPT // LANES):
        zvec_v[pl.ds(i * LANES, LANES)] = jnp.zeros((LANES,), jnp.float32)
    pltpu.sync_copy(zvec_v, cnt_sh.at[pl.ds(s * RPT, RPT)])
    ld.wait()
    plsc.subcore_barrier()
    descs = [
        pltpu.async_copy(ones_v, cnt_sh.at[idx_v.at[j]], sem, add=True)
        for j in range(NB)
    ]
    for dd in descs:
        dd.wait()
    plsc.subcore_barrier()
    pltpu.sync_copy(cnt_sh.at[pl.ds(s * RPT, RPT)],
                    cnt_hbm.at[c, pl.ds(s * RPT, RPT)])


_count_edges = functools.partial(
    pl.kernel,
    _count_body,
    out_type=jax.ShapeDtypeStruct((NC, NPAD), jnp.float32),
    mesh=_MESH,
    scratch_types=[
        pltpu.VMEM_SHARED((NPAD,), jnp.float32),
        pltpu.VMEM((NB, BATCH), jnp.int32),
        pltpu.VMEM((BATCH,), jnp.float32),
        pltpu.VMEM((RPT,), jnp.float32),
        pltpu.SemaphoreType.DMA,
    ],
    compiler_params=_SC_PARAMS,
)()


def _scatter_body(eip_hbm, y_hbm, agg_hbm,
                  agg_sh, isrc_v, idst_v, rows_v, zbuf_v, gsem, ssem):
    c = lax.axis_index("c")
    s = lax.axis_index("s")
    wid = c * NS + s
    lds = pltpu.async_copy(eip_hbm.at[0, pl.ds(wid * NB, NB)], isrc_v, gsem)
    ldd = pltpu.async_copy(eip_hbm.at[1, pl.ds(wid * NB, NB)], idst_v, ssem)
    for r in range(ZR):
        for k in range(H // LANES):
            zbuf_v[r, pl.ds(k * LANES, LANES)] = jnp.zeros((LANES,), jnp.float32)
    for t in range(RPT // ZR):
        pltpu.sync_copy(zbuf_v, agg_sh.at[pl.ds(s * RPT + t * ZR, ZR)])
    plsc.subcore_barrier()
    lds.wait()
    ldd.wait()
    ncH = NB // CK

    def _fire_gather(t, buf):
        return [
            pltpu.async_copy(y_hbm.at[isrc_v.at[CK * t + j]],
                             rows_v.at[buf, j], gsem)
            for j in range(CK)
        ]

    def _fire_scatter(t, buf):
        return [
            pltpu.async_copy(rows_v.at[buf, j],
                             agg_sh.at[idst_v.at[CK * t + j]], ssem, add=True)
            for j in range(CK)
        ]

    gd = _fire_gather(0, 0)
    sd = []
    for t in range(ncH):
        p = t % 2
        for dd in gd:
            dd.wait()
        for dd in sd:
            dd.wait()
        gd = _fire_gather(t + 1, 1 - p) if t + 1 < ncH else []
        sd = _fire_scatter(t, p)
    for dd in sd:
        dd.wait()
    plsc.subcore_barrier()
    pltpu.sync_copy(agg_sh.at[pl.ds(s * RPT, RPT)],
                    agg_hbm.at[c, pl.ds(s * RPT, RPT)])


_scatter_edges = functools.partial(
    pl.kernel,
    _scatter_body,
    out_type=jax.ShapeDtypeStruct((NC, NPAD, H), jnp.float32),
    mesh=_MESH,
    scratch_types=[
        pltpu.VMEM_SHARED((NPAD, H), jnp.float32),
        pltpu.VMEM((NB, BATCH), jnp.int32),
        pltpu.VMEM((NB, BATCH), jnp.int32),
        pltpu.VMEM((2, CK, BATCH, H), jnp.float32),
        pltpu.VMEM((ZR, H), jnp.float32),
        pltpu.SemaphoreType.DMA,
        pltpu.SemaphoreType.DMA,
    ],
    compiler_params=_SC_PARAMS,
)()


BM = 2560


def _y1_body(x_ref, w_ref, cnt_ref, y_ref):
    dinv = lax.rsqrt(1.0 + cnt_ref[0, :] + cnt_ref[1, :])
    xw = jnp.dot(x_ref[...], w_ref[...], preferred_element_type=jnp.float32)
    y_ref[...] = xw * dinv[:, None]


def _mid_body(agg_ref, y_ref, cnt_ref, b_ref, w_ref, out_ref):
    dinv = lax.rsqrt(1.0 + cnt_ref[0, :] + cnt_ref[1, :])
    pre = (agg_ref[0] + agg_ref[1] + y_ref[...]) * dinv[:, None] + b_ref[...]
    h = jnp.maximum(pre, 0.0)
    hw = jnp.dot(h, w_ref[...], preferred_element_type=jnp.float32)
    out_ref[...] = hw * dinv[:, None]


def _fin_body(agg_ref, y_ref, cnt_ref, b_ref, wfc_ref, bfc_ref, out_ref):
    dinv = lax.rsqrt(1.0 + cnt_ref[0, :] + cnt_ref[1, :])
    pre = (agg_ref[0] + agg_ref[1] + y_ref[...]) * dinv[:, None] + b_ref[...]
    h = jnp.maximum(pre, 0.0)
    out_ref[...] = (jnp.dot(h, wfc_ref[...], preferred_element_type=jnp.float32)
                    + bfc_ref[...])


def _tc_y1(x, W1, cnt):
    return pl.pallas_call(
        _y1_body,
        grid=(NPAD // BM,),
        in_specs=[
            pl.BlockSpec((BM, D), lambda i: (i, 0)),
            pl.BlockSpec((D, H), lambda i: (0, 0)),
            pl.BlockSpec((NC, BM), lambda i: (0, i)),
        ],
        out_specs=pl.BlockSpec((BM, H), lambda i: (i, 0)),
        out_shape=jax.ShapeDtypeStruct((N, H), jnp.float32),
    )(x, W1, cnt)


def _tc_mid(agg, y1, cnt, b1, W2):
    return pl.pallas_call(
        _mid_body,
        grid=(NPAD // BM,),
        in_specs=[
            pl.BlockSpec((NC, BM, H), lambda i: (0, i, 0)),
            pl.BlockSpec((BM, H), lambda i: (i, 0)),
            pl.BlockSpec((NC, BM), lambda i: (0, i)),
            pl.BlockSpec((1, H), lambda i: (0, 0)),
            pl.BlockSpec((H, H), lambda i: (0, 0)),
        ],
        out_specs=pl.BlockSpec((BM, H), lambda i: (i, 0)),
        out_shape=jax.ShapeDtypeStruct((N, H), jnp.float32),
    )(agg, y1, cnt, b1.reshape(1, H), W2)


def _tc_fin(agg, y2, cnt, b2, Wfc, bfc):
    return pl.pallas_call(
        _fin_body,
        grid=(NPAD // BM,),
        in_specs=[
            pl.BlockSpec((NC, BM, H), lambda i: (0, i, 0)),
            pl.BlockSpec((BM, H), lambda i: (i, 0)),
            pl.BlockSpec((NC, BM), lambda i: (0, i)),
            pl.BlockSpec((1, H), lambda i: (0, 0)),
            pl.BlockSpec((H, H), lambda i: (0, 0)),
            pl.BlockSpec((1, H), lambda i: (0, 0)),
        ],
        out_specs=pl.BlockSpec((BM, H), lambda i: (i, 0)),
        out_shape=jax.ShapeDtypeStruct((N, H), jnp.float32),
    )(agg, y2, cnt, b2.reshape(1, H), Wfc, bfc.reshape(1, H))


def kernel(x, edge_index, W1, b1, W2, b2, Wfc, bfc):
    ei = edge_index.astype(jnp.int32)
    spread = jnp.arange(EPAD - E, dtype=jnp.int32) % NJUNK
    pad2 = jnp.stack([spread, N + spread])
    eip = jnp.concatenate([ei, pad2], axis=1).reshape(2, EPAD // BATCH, BATCH)

    cnt = _count_edges(eip)
    y1 = _tc_y1(x, W1, cnt)
    agg1 = _scatter_edges(eip, y1)
    y2 = _tc_mid(agg1, y1, cnt, b1, W2)
    agg2 = _scatter_edges(eip, y2)
    return _tc_fin(agg2, y2, cnt, b2, Wfc, bfc)
